# Initial kernel scaffold; baseline (speedup 1.0000x reference)
import jax
import jax.numpy as jnp
from jax.experimental import pallas as pl


def kernel(vol, phi):
    raise NotImplementedError("write your pallas kernel here")


# speedup vs baseline: 1.7304x; 1.7304x over previous
import functools

import jax
import jax.numpy as jnp
from jax import lax
from jax.experimental import pallas as pl
from jax.experimental.pallas import tpu as pltpu
from jax.experimental.pallas import tpu_sc as plsc

_LANES = 16


@functools.lru_cache(maxsize=None)
def _make_warp(B, C, D, H, W):
    assert C == 1
    N = D * H * W
    TOTAL = B * N
    HW = H * W
    info = plsc.get_sparse_core_info()
    NW = info.num_cores * info.num_subcores
    wpb = NW // B
    per_w = N // wpb
    CH = 4096
    n_chunks = per_w // CH
    assert W & (W - 1) == 0 and H & (H - 1) == 0
    w_sh = W.bit_length() - 1
    hw_sh = HW.bit_length() - 1
    offs = [0, 1, W, W + 1, HW, HW + 1, HW + W, HW + W + 1]

    mesh = plsc.VectorSubcoreMesh(core_axis_name="c", subcore_axis_name="s")

    @functools.partial(
        pl.kernel,
        mesh=mesh,
        out_type=jax.ShapeDtypeStruct((TOTAL,), jnp.float32),
        scratch_types=(
            [pltpu.VMEM((CH,), jnp.float32)] * 3
            + [pltpu.VMEM((CH,), jnp.int32)] * 8
            + [pltpu.VMEM((CH,), jnp.float32)] * 8
            + [pltpu.VMEM((CH,), jnp.float32)]
            + [pltpu.SemaphoreType.DMA]
        ),
    )
    def warp(vol_hbm, phi_hbm, out_hbm, *scratch):
        wbuf = scratch[0:3]
        ibuf = scratch[3:11]
        gbuf = scratch[11:19]
        obuf = scratch[19]
        sem = scratch[20]
        wid = lax.axis_index("c") * info.num_subcores + lax.axis_index("s")
        b = wid // wpb
        s0 = (wid % wpb) * per_w
        vi = lax.iota(jnp.int32, _LANES)

        def chunk_body(ci, _):
            sbase = s0 + ci * CH
            for ch3 in range(3):
                pltpu.sync_copy(
                    phi_hbm.at[pl.ds((b * 3 + ch3) * N + sbase, CH)],
                    wbuf[ch3],
                )

            def grp(g, _):
                o = g * _LANES
                s = sbase + o + vi
                wg = s & (W - 1)
                hg = (s >> w_sh) & (H - 1)
                dg = s >> hw_sh

                def corner(grid_i, pval, lim):
                    cf = grid_i.astype(jnp.float32) + pval
                    cf = jnp.minimum(jnp.maximum(cf, 0.0), float(lim - 1))
                    i0 = jnp.minimum(cf.astype(jnp.int32), lim - 2)
                    return i0, cf - i0.astype(jnp.float32)

                d0, wd = corner(dg, wbuf[0][pl.ds(o, _LANES)], D)
                h0, wh = corner(hg, wbuf[1][pl.ds(o, _LANES)], H)
                w0, ww = corner(wg, wbuf[2][pl.ds(o, _LANES)], W)
                idx0 = b * N + (d0 * H + h0) * W + w0
                for k in range(8):
                    ibuf[k][pl.ds(o, _LANES)] = idx0 + offs[k]
                wbuf[0][pl.ds(o, _LANES)] = wd
                wbuf[1][pl.ds(o, _LANES)] = wh
                wbuf[2][pl.ds(o, _LANES)] = ww

            lax.fori_loop(0, CH // _LANES, grp, None)

            copies = [
                pltpu.async_copy(vol_hbm.at[ibuf[k]], gbuf[k], sem)
                for k in range(8)
            ]
            for cp in copies:
                cp.wait()

            def blend(g, _):
                o = g * _LANES
                c = [gbuf[k][pl.ds(o, _LANES)] for k in range(8)]
                wd = wbuf[0][pl.ds(o, _LANES)]
                wh = wbuf[1][pl.ds(o, _LANES)]
                ww = wbuf[2][pl.ds(o, _LANES)]
                a00 = c[0] + ww * (c[1] - c[0])
                a01 = c[2] + ww * (c[3] - c[2])
                a10 = c[4] + ww * (c[5] - c[4])
                a11 = c[6] + ww * (c[7] - c[6])
                b0 = a00 + wh * (a01 - a00)
                b1 = a10 + wh * (a11 - a10)
                obuf[pl.ds(o, _LANES)] = b0 + wd * (b1 - b0)

            lax.fori_loop(0, CH // _LANES, blend, None)
            pltpu.sync_copy(obuf, out_hbm.at[pl.ds(b * N + sbase, CH)])

        lax.fori_loop(0, n_chunks, chunk_body, None)

    return warp


def kernel(vol, phi):
    B, C, D, H, W = vol.shape
    warp = _make_warp(B, C, D, H, W)
    out = warp(vol.reshape(-1), phi.reshape(-1))
    return out.reshape(B, C, D, H, W)


# speedup vs baseline: 1.8441x; 1.0657x over previous
import functools

import jax
import jax.numpy as jnp
from jax import lax
from jax.experimental import pallas as pl
from jax.experimental.pallas import tpu as pltpu
from jax.experimental.pallas import tpu_sc as plsc

_LANES = 16


@functools.lru_cache(maxsize=None)
def _make_warp(B, C, D, H, W):
    assert C == 1
    N = D * H * W
    TOTAL = B * N
    HW = H * W
    info = plsc.get_sparse_core_info()
    NW = info.num_cores * info.num_subcores
    wpb = NW // B
    per_w = N // wpb
    CH = 2048
    n_pairs = per_w // (2 * CH)
    assert W & (W - 1) == 0 and H & (H - 1) == 0
    w_sh = W.bit_length() - 1
    hw_sh = HW.bit_length() - 1
    offs = [0, 1, W, W + 1, HW, HW + 1, HW + W, HW + W + 1]

    mesh = plsc.VectorSubcoreMesh(core_axis_name="c", subcore_axis_name="s")

    _set = (
        [pltpu.VMEM((CH,), jnp.float32)] * 3
        + [pltpu.VMEM((CH,), jnp.int32)] * 8
        + [pltpu.VMEM((CH,), jnp.float32)] * 8
    )

    @functools.partial(
        pl.kernel,
        mesh=mesh,
        out_type=jax.ShapeDtypeStruct((TOTAL,), jnp.float32),
        scratch_types=(
            _set + _set
            + [pltpu.VMEM((CH,), jnp.float32)]
            + [pltpu.SemaphoreType.DMA] * 2
        ),
    )
    def warp(vol_hbm, phi_hbm, out_hbm, *scratch):
        sets = (scratch[0:19], scratch[19:38])
        obuf = scratch[38]
        sems = scratch[39:41]
        wid = lax.axis_index("c") * info.num_subcores + lax.axis_index("s")
        b = wid // wpb
        s0 = (wid % wpb) * per_w
        vi = lax.iota(jnp.int32, _LANES)

        def fire(sbase, bufs, sem):
            wbuf, ibuf = bufs[0:3], bufs[3:11]
            for ch3 in range(3):
                pltpu.sync_copy(
                    phi_hbm.at[pl.ds((b * 3 + ch3) * N + sbase, CH)],
                    wbuf[ch3],
                )

            def grp(g, _):
                o = g * _LANES
                s = sbase + o + vi
                wg = s & (W - 1)
                hg = (s >> w_sh) & (H - 1)
                dg = s >> hw_sh

                def corner(grid_i, pval, lim):
                    cf = grid_i.astype(jnp.float32) + pval
                    cf = jnp.minimum(jnp.maximum(cf, 0.0), float(lim - 1))
                    i0 = jnp.minimum(cf.astype(jnp.int32), lim - 2)
                    return i0, cf - i0.astype(jnp.float32)

                d0, wd = corner(dg, wbuf[0][pl.ds(o, _LANES)], D)
                h0, wh = corner(hg, wbuf[1][pl.ds(o, _LANES)], H)
                w0, ww = corner(wg, wbuf[2][pl.ds(o, _LANES)], W)
                idx0 = b * N + (d0 * H + h0) * W + w0
                for k in range(8):
                    ibuf[k][pl.ds(o, _LANES)] = idx0 + offs[k]
                wbuf[0][pl.ds(o, _LANES)] = wd
                wbuf[1][pl.ds(o, _LANES)] = wh
                wbuf[2][pl.ds(o, _LANES)] = ww

            lax.fori_loop(0, CH // _LANES, grp, None)
            return [
                pltpu.async_copy(vol_hbm.at[ibuf[k]], bufs[11 + k], sem)
                for k in range(8)
            ]

        def drain(sbase, bufs, copies):
            wbuf, gbuf = bufs[0:3], bufs[11:19]
            for cp in copies:
                cp.wait()

            def blend(g, _):
                o = g * _LANES
                c = [gbuf[k][pl.ds(o, _LANES)] for k in range(8)]
                wd = wbuf[0][pl.ds(o, _LANES)]
                wh = wbuf[1][pl.ds(o, _LANES)]
                ww = wbuf[2][pl.ds(o, _LANES)]
                a00 = c[0] + ww * (c[1] - c[0])
                a01 = c[2] + ww * (c[3] - c[2])
                a10 = c[4] + ww * (c[5] - c[4])
                a11 = c[6] + ww * (c[7] - c[6])
                b0 = a00 + wh * (a01 - a00)
                b1 = a10 + wh * (a11 - a10)
                obuf[pl.ds(o, _LANES)] = b0 + wd * (b1 - b0)

            lax.fori_loop(0, CH // _LANES, blend, None)
            pltpu.sync_copy(obuf, out_hbm.at[pl.ds(b * N + sbase, CH)])

        def pair_body(pi, _):
            sa = s0 + pi * 2 * CH
            sb = sa + CH
            ca = fire(sa, sets[0], sems[0])
            cb = fire(sb, sets[1], sems[1])
            drain(sa, sets[0], ca)
            drain(sb, sets[1], cb)

        lax.fori_loop(0, n_pairs, pair_body, None)

    return warp


def kernel(vol, phi):
    B, C, D, H, W = vol.shape
    warp = _make_warp(B, C, D, H, W)
    out = warp(vol.reshape(-1), phi.reshape(-1))
    return out.reshape(B, C, D, H, W)


# speedup vs baseline: 1.8551x; 1.0059x over previous
import functools

import jax
import jax.numpy as jnp
from jax import lax
from jax.experimental import pallas as pl
from jax.experimental.pallas import tpu as pltpu
from jax.experimental.pallas import tpu_sc as plsc

_LANES = 16


@functools.lru_cache(maxsize=None)
def _make_warp(B, C, D, H, W):
    assert C == 1
    N = D * H * W
    TOTAL = B * N
    HW = H * W
    info = plsc.get_sparse_core_info()
    NW = info.num_cores * info.num_subcores
    wpb = NW // B
    per_w = N // wpb
    CH = 2048
    n_pairs = per_w // (2 * CH)
    assert W & (W - 1) == 0 and H & (H - 1) == 0
    w_sh = W.bit_length() - 1
    hw_sh = HW.bit_length() - 1
    offs = [0, 1, W, W + 1, HW, HW + 1, HW + W, HW + W + 1]

    mesh = plsc.VectorSubcoreMesh(core_axis_name="c", subcore_axis_name="s")

    _set = (
        [pltpu.VMEM((CH,), jnp.float32)] * 3
        + [pltpu.VMEM((CH,), jnp.int32)] * 8
        + [pltpu.VMEM((CH,), jnp.float32)] * 8
    )

    @functools.partial(
        pl.kernel,
        mesh=mesh,
        out_type=jax.ShapeDtypeStruct((TOTAL,), jnp.float32),
        scratch_types=(
            _set + _set
            + [pltpu.VMEM((CH,), jnp.float32)]
            + [pltpu.SemaphoreType.DMA] * 2
        ),
    )
    def warp(vol_hbm, phi_hbm, out_hbm, *scratch):
        sets = (scratch[0:19], scratch[19:38])
        obuf = scratch[38]
        sems = scratch[39:41]
        wid = lax.axis_index("c") * info.num_subcores + lax.axis_index("s")
        b = wid // wpb
        s0 = (wid % wpb) * per_w
        vi = lax.iota(jnp.int32, _LANES)

        def fire(sbase, bufs, sem):
            wbuf, ibuf = bufs[0:3], bufs[3:11]
            for ch3 in range(3):
                pltpu.sync_copy(
                    phi_hbm.at[pl.ds((b * 3 + ch3) * N + sbase, CH)],
                    wbuf[ch3],
                )

            @plsc.parallel_loop(0, CH // _LANES, unroll=4)
            def grp(g):
                o = g * _LANES
                s = sbase + o + vi
                wg = s & (W - 1)
                hg = (s >> w_sh) & (H - 1)
                dg = s >> hw_sh

                def corner(grid_i, pval, lim):
                    cf = grid_i.astype(jnp.float32) + pval
                    cf = jnp.minimum(jnp.maximum(cf, 0.0), float(lim - 1))
                    i0 = jnp.minimum(cf.astype(jnp.int32), lim - 2)
                    return i0, cf - i0.astype(jnp.float32)

                d0, wd = corner(dg, wbuf[0][pl.ds(o, _LANES)], D)
                h0, wh = corner(hg, wbuf[1][pl.ds(o, _LANES)], H)
                w0, ww = corner(wg, wbuf[2][pl.ds(o, _LANES)], W)
                idx0 = b * N + (d0 * H + h0) * W + w0
                for k in range(8):
                    ibuf[k][pl.ds(o, _LANES)] = idx0 + offs[k]
                wbuf[0][pl.ds(o, _LANES)] = wd
                wbuf[1][pl.ds(o, _LANES)] = wh
                wbuf[2][pl.ds(o, _LANES)] = ww

            return [
                pltpu.async_copy(vol_hbm.at[ibuf[k]], bufs[11 + k], sem)
                for k in range(8)
            ]

        def drain(sbase, bufs, copies):
            wbuf, gbuf = bufs[0:3], bufs[11:19]
            for cp in copies:
                cp.wait()

            @plsc.parallel_loop(0, CH // _LANES, unroll=4)
            def blend(g):
                o = g * _LANES
                c = [gbuf[k][pl.ds(o, _LANES)] for k in range(8)]
                wd = wbuf[0][pl.ds(o, _LANES)]
                wh = wbuf[1][pl.ds(o, _LANES)]
                ww = wbuf[2][pl.ds(o, _LANES)]
                a00 = c[0] + ww * (c[1] - c[0])
                a01 = c[2] + ww * (c[3] - c[2])
                a10 = c[4] + ww * (c[5] - c[4])
                a11 = c[6] + ww * (c[7] - c[6])
                b0 = a00 + wh * (a01 - a00)
                b1 = a10 + wh * (a11 - a10)
                obuf[pl.ds(o, _LANES)] = b0 + wd * (b1 - b0)

            pltpu.sync_copy(obuf, out_hbm.at[pl.ds(b * N + sbase, CH)])

        def pair_body(pi, _):
            sa = s0 + pi * 2 * CH
            sb = sa + CH
            ca = fire(sa, sets[0], sems[0])
            cb = fire(sb, sets[1], sems[1])
            drain(sa, sets[0], ca)
            drain(sb, sets[1], cb)

        lax.fori_loop(0, n_pairs, pair_body, None)

    return warp


def kernel(vol, phi):
    B, C, D, H, W = vol.shape
    warp = _make_warp(B, C, D, H, W)
    out = warp(vol.reshape(-1), phi.reshape(-1))
    return out.reshape(B, C, D, H, W)


# speedup vs baseline: 1.9002x; 1.0243x over previous
import functools

import jax
import jax.numpy as jnp
from jax import lax
from jax.experimental import pallas as pl
from jax.experimental.pallas import tpu as pltpu
from jax.experimental.pallas import tpu_sc as plsc

_LANES = 16


@functools.lru_cache(maxsize=None)
def _make_warp(B, C, D, H, W):
    assert C == 1
    N = D * H * W
    TOTAL = B * N
    HW = H * W
    info = plsc.get_sparse_core_info()
    NW = info.num_cores * info.num_subcores
    wpb = NW // B
    per_w = N // wpb
    CH = 2048
    n_pairs = per_w // (2 * CH)
    assert W & (W - 1) == 0 and H & (H - 1) == 0
    w_sh = W.bit_length() - 1
    hw_sh = HW.bit_length() - 1
    offs = [0, 1, W, W + 1, HW, HW + 1, HW + W, HW + W + 1]

    mesh = plsc.VectorSubcoreMesh(core_axis_name="c", subcore_axis_name="s")

    _set = (
        [pltpu.VMEM((CH,), jnp.float32)] * 3
        + [pltpu.VMEM((CH,), jnp.int32)] * 8
        + [pltpu.VMEM((CH,), jnp.float32)] * 8
    )

    @functools.partial(
        pl.kernel,
        mesh=mesh,
        out_type=jax.ShapeDtypeStruct((TOTAL,), jnp.float32),
        scratch_types=(
            _set + _set
            + [pltpu.VMEM((CH,), jnp.float32)]
            + [pltpu.SemaphoreType.DMA] * 2
        ),
    )
    def warp(vol_hbm, phi_hbm, out_hbm, *scratch):
        sets = (scratch[0:19], scratch[19:38])
        obuf = scratch[38]
        sems = scratch[39:41]
        wid = lax.axis_index("c") * info.num_subcores + lax.axis_index("s")
        b = wid // wpb
        s0 = (wid % wpb) * per_w
        vi = lax.iota(jnp.int32, _LANES)

        def fire(sbase, bufs, sem):
            wbuf, ibuf = bufs[0:3], bufs[3:11]
            for ch3 in range(3):
                pltpu.sync_copy(
                    phi_hbm.at[pl.ds((b * 3 + ch3) * N + sbase, CH)],
                    wbuf[ch3],
                )

            @plsc.parallel_loop(0, CH // _LANES, unroll=4)
            def grp(g):
                o = g * _LANES
                s = sbase + o + vi
                wg = s & (W - 1)
                hg = (s >> w_sh) & (H - 1)
                dg = s >> hw_sh

                def corner(grid_i, pval, lim):
                    cf = grid_i.astype(jnp.float32) + pval
                    cf = jnp.minimum(jnp.maximum(cf, 0.0), float(lim - 1))
                    i0 = jnp.minimum(cf.astype(jnp.int32), lim - 2)
                    return i0, cf - i0.astype(jnp.float32)

                d0, wd = corner(dg, wbuf[0][pl.ds(o, _LANES)], D)
                h0, wh = corner(hg, wbuf[1][pl.ds(o, _LANES)], H)
                w0, ww = corner(wg, wbuf[2][pl.ds(o, _LANES)], W)
                idx0 = b * N + (d0 * H + h0) * W + w0
                for k in range(8):
                    ibuf[k][pl.ds(o, _LANES)] = idx0 + offs[k]
                wbuf[0][pl.ds(o, _LANES)] = wd
                wbuf[1][pl.ds(o, _LANES)] = wh
                wbuf[2][pl.ds(o, _LANES)] = ww

            for k in range(8):
                pltpu.async_copy(vol_hbm.at[ibuf[k]], bufs[11 + k], sem)

        def drain(sbase, bufs, sem):
            wbuf, ibuf, gbuf = bufs[0:3], bufs[3:11], bufs[11:19]
            for k in range(8):
                pltpu.make_async_copy(
                    vol_hbm.at[ibuf[k]], gbuf[k], sem
                ).wait()

            @plsc.parallel_loop(0, CH // _LANES, unroll=4)
            def blend(g):
                o = g * _LANES
                c = [gbuf[k][pl.ds(o, _LANES)] for k in range(8)]
                wd = wbuf[0][pl.ds(o, _LANES)]
                wh = wbuf[1][pl.ds(o, _LANES)]
                ww = wbuf[2][pl.ds(o, _LANES)]
                a00 = c[0] + ww * (c[1] - c[0])
                a01 = c[2] + ww * (c[3] - c[2])
                a10 = c[4] + ww * (c[5] - c[4])
                a11 = c[6] + ww * (c[7] - c[6])
                b0 = a00 + wh * (a01 - a00)
                b1 = a10 + wh * (a11 - a10)
                obuf[pl.ds(o, _LANES)] = b0 + wd * (b1 - b0)

            pltpu.sync_copy(obuf, out_hbm.at[pl.ds(b * N + sbase, CH)])

        def pair_body(pi, _):
            sa = s0 + pi * 2 * CH
            sb = sa + CH
            fire(sb, sets[1], sems[1])
            drain(sa, sets[0], sems[0])

            @pl.when(pi + 1 < n_pairs)
            def _():
                fire(sa + 2 * CH, sets[0], sems[0])

            drain(sb, sets[1], sems[1])

        fire(s0, sets[0], sems[0])
        lax.fori_loop(0, n_pairs, pair_body, None)

    return warp


def kernel(vol, phi):
    B, C, D, H, W = vol.shape
    warp = _make_warp(B, C, D, H, W)
    out = warp(vol.reshape(-1), phi.reshape(-1))
    return out.reshape(B, C, D, H, W)
